# baseline (speedup 1.0000x reference)
import functools
import math

import jax
import jax.numpy as jnp
from jax import lax
from jax.experimental import pallas as pl
from jax.experimental.pallas import tpu as pltpu
from jax.experimental.pallas import tpu_sc as plsc

D_MODEL = 1024
SCALE = float(math.sqrt(D_MODEL))

NC = 2
NS = 16
NW = NC * NS
LANES = 16

B_TOTAL = 4 * 4096
BPW = B_TOTAL // NW
CHUNK = 16
NCH = BPW // CHUNK
NBUF = 4
NGRP = NCH // NBUF
assert NCH % NBUF == 0 and NGRP >= 3


def _emb_body(x_hbm, table_hbm, out_hbm, idx_v, rows_v, *sems):
    gsems = sems[:NBUF]
    osems = sems[NBUF:]
    wid = lax.axis_index("s") * NC + lax.axis_index("c")
    base = wid * BPW

    pltpu.sync_copy(x_hbm.at[pl.ds(base, BPW)], idx_v)

    def start_gather(c, b):
        return pltpu.async_copy(
            table_hbm.at[idx_v.at[pl.ds(c * CHUNK, CHUNK)]],
            rows_v.at[b], gsems[b])

    def wait_gather(c, b):
        pltpu.make_async_copy(
            table_hbm.at[idx_v.at[pl.ds(c * CHUNK, CHUNK)]],
            rows_v.at[b], gsems[b]).wait()

    def start_scatter(c, b):
        return pltpu.async_copy(
            rows_v.at[b], out_hbm.at[pl.ds(base + c * CHUNK, CHUNK)],
            osems[b])

    def wait_scatter(c, b):
        pltpu.make_async_copy(
            rows_v.at[b], out_hbm.at[pl.ds(base + c * CHUNK, CHUNK)],
            osems[b]).wait()

    def scale(b):
        @pl.loop(0, CHUNK)
        def _scale_row(i):
            for j in range(D_MODEL // LANES):
                sl = pl.ds(j * LANES, LANES)
                rows_v[b, i, sl] = rows_v[b, i, sl] * SCALE

    def step(c, b, first, last):
        bn = (b + NBUF - 1) % NBUF
        if not last:
            if not first:
                wait_scatter(c - 1, bn)
            start_gather(c + NBUF - 1, bn)
        wait_gather(c, b)
        scale(b)
        start_scatter(c, b)

    for b in range(NBUF - 1):
        start_gather(b, b)

    for c in range(NCH):
        step(c, c % NBUF, first=(c == 0), last=(c + NBUF - 1 >= NCH))

    for c in range(NCH - NBUF, NCH):
        wait_scatter(c, c % NBUF)


def _emb(x_flat, table):
    f = functools.partial(
        pl.kernel,
        out_type=jax.ShapeDtypeStruct((B_TOTAL, D_MODEL), jnp.float32),
        mesh=plsc.VectorSubcoreMesh(
            core_axis_name="c", subcore_axis_name="s",
            num_cores=NC, num_subcores=NS),
        scratch_types=[
            pltpu.VMEM((BPW,), jnp.int32),
            pltpu.VMEM((NBUF, CHUNK, D_MODEL), jnp.float32),
        ] + [pltpu.SemaphoreType.DMA] * (2 * NBUF),
    )(_emb_body)
    return f(x_flat, table)


def kernel(x, table):
    xf = x.reshape(-1).astype(jnp.int32)
    out = _emb(xf, table)
    return out.reshape(x.shape + (D_MODEL,))


# speedup vs baseline: 1.1750x; 1.1750x over previous
import functools
import math

import jax
import jax.numpy as jnp
from jax import lax
from jax.experimental import pallas as pl
from jax.experimental.pallas import tpu as pltpu
from jax.experimental.pallas import tpu_sc as plsc

D_MODEL = 1024
SCALE = float(math.sqrt(D_MODEL))

NC = 2
NS = 16
NW = NC * NS
LANES = 16

B_TOTAL = 4 * 4096
BPW = B_TOTAL // NW
CHUNK = 32
NCH = BPW // CHUNK
NBUF = 3
assert NCH >= NBUF


def _emb_body(x_hbm, table_hbm, out_hbm, idx_v, rows_v, *sems):
    gsems = sems[:NBUF]
    osems = sems[NBUF:]
    wid = lax.axis_index("s") * NC + lax.axis_index("c")
    base = wid * BPW

    pltpu.sync_copy(x_hbm.at[pl.ds(base, BPW)], idx_v)

    def start_gather(c, b):
        return pltpu.async_copy(
            table_hbm.at[idx_v.at[pl.ds(c * CHUNK, CHUNK)]],
            rows_v.at[b], gsems[b])

    def wait_gather(c, b):
        pltpu.make_async_copy(
            table_hbm.at[idx_v.at[pl.ds(c * CHUNK, CHUNK)]],
            rows_v.at[b], gsems[b]).wait()

    def start_scatter(c, b):
        return pltpu.async_copy(
            rows_v.at[b], out_hbm.at[pl.ds(base + c * CHUNK, CHUNK)],
            osems[b])

    def wait_scatter(c, b):
        pltpu.make_async_copy(
            rows_v.at[b], out_hbm.at[pl.ds(base + c * CHUNK, CHUNK)],
            osems[b]).wait()

    def scale(b):
        @pl.loop(0, CHUNK)
        def _scale_row(i):
            for j in range(D_MODEL // LANES):
                sl = pl.ds(j * LANES, LANES)
                rows_v[b, i, sl] = rows_v[b, i, sl] * SCALE

    def step(c, b, first, last):
        bn = (b + NBUF - 1) % NBUF
        if not last:
            if not first:
                wait_scatter(c - 1, bn)
            start_gather(c + NBUF - 1, bn)
        wait_gather(c, b)
        scale(b)
        start_scatter(c, b)

    for b in range(NBUF - 1):
        start_gather(b, b)

    for c in range(NCH):
        step(c, c % NBUF, first=(c == 0), last=(c + NBUF - 1 >= NCH))

    for c in range(NCH - NBUF, NCH):
        wait_scatter(c, c % NBUF)


def _emb(x_flat, table):
    f = functools.partial(
        pl.kernel,
        out_type=jax.ShapeDtypeStruct((B_TOTAL, D_MODEL), jnp.float32),
        mesh=plsc.VectorSubcoreMesh(
            core_axis_name="c", subcore_axis_name="s",
            num_cores=NC, num_subcores=NS),
        scratch_types=[
            pltpu.VMEM((BPW,), jnp.int32),
            pltpu.VMEM((NBUF, CHUNK, D_MODEL), jnp.float32),
        ] + [pltpu.SemaphoreType.DMA] * (2 * NBUF),
    )(_emb_body)
    return f(x_flat, table)


def kernel(x, table):
    xf = x.reshape(-1).astype(jnp.int32)
    out = _emb(xf, table)
    return out.reshape(x.shape + (D_MODEL,))
